# Initial kernel scaffold; baseline (speedup 1.0000x reference)
import functools

import jax
import jax.numpy as jnp
from jax import lax
from jax.experimental import pallas as pl
from jax.experimental.pallas import tpu as pltpu
from jax.experimental.pallas import tpu_sc as plsc

_NC = 2
_NS = 16
_NW = _NC * _NS
_L = 16

_N = 50000
_E = 800000
_GN = 1568
_NPAD = _NW * _GN
_EW = _E // _NW
_C = 1000
_NCHUNK = _EW // _C


def _sc_body(w_hbm, x_hbm, e0_hbm, e1_hbm, nv_hbm, st_hbm,
             wv, xv, nvf, idx0, idx1, r0, r1, obf, sem0, sem1):
  wid = lax.axis_index("s") * _NC + lax.axis_index("c")
  lane = lax.iota(jnp.int32, _L)

  nbase = wid * _GN
  pltpu.sync_copy(w_hbm.at[pl.ds(nbase, _GN)], wv)
  pltpu.sync_copy(x_hbm.at[pl.ds(nbase, _GN)], xv)

  def nv_group(g, carry):
    nid = g * _L + lane
    xs = [plsc.load_gather(xv, [nid, jnp.full((_L,), j, jnp.int32)])
          for j in range(3)]
    for i in range(3):
      acc = plsc.load_gather(wv, [nid, jnp.full((_L,), 4 * i + 3, jnp.int32)])
      for j in range(3):
        wij = plsc.load_gather(wv, [nid, jnp.full((_L,), 4 * i + j, jnp.int32)])
        acc = acc + wij * xs[j]
      plsc.store_scatter(nvf, [nid * 3 + i], acc)
    return carry

  lax.fori_loop(0, _GN // _L, nv_group, 0)
  pltpu.sync_copy(nvf, nv_hbm.at[pl.ds(nbase * 3, _GN * 3)])

  ebase = wid * _EW

  def chunk(k, carry):
    cb = ebase + k * _C
    pltpu.sync_copy(e0_hbm.at[pl.ds(cb, _C)], idx0)
    pltpu.sync_copy(e1_hbm.at[pl.ds(cb, _C)], idx1)
    cp0 = pltpu.async_copy(w_hbm.at[idx0], r0, sem0)
    cp1 = pltpu.async_copy(w_hbm.at[idx1], r1, sem1)
    cp0.wait()
    cp1.wait()

    def row(c, rcarry):
      d = r0[c] - r1[c]
      plsc.store_scatter(obf, [c * 12 + lane], d * d, mask=lane < 12)
      return rcarry

    lax.fori_loop(0, _C, row, 0)
    pltpu.sync_copy(obf, st_hbm.at[pl.ds(cb * 12, _C * 12)])
    return carry

  lax.fori_loop(0, _NCHUNK, chunk, 0)


_sc_kernel = functools.partial(
    pl.kernel,
    out_type=(
        jax.ShapeDtypeStruct((_NPAD * 3,), jnp.float32),
        jax.ShapeDtypeStruct((_E * 12,), jnp.float32),
    ),
    mesh=plsc.VectorSubcoreMesh(
        core_axis_name="c", subcore_axis_name="s",
        num_cores=_NC, num_subcores=_NS),
    scratch_types=[
        pltpu.VMEM((_GN, 16), jnp.float32),
        pltpu.VMEM((_GN, 3), jnp.float32),
        pltpu.VMEM((_GN * 3,), jnp.float32),
        pltpu.VMEM((_C,), jnp.int32),
        pltpu.VMEM((_C,), jnp.int32),
        pltpu.VMEM((_C, 16), jnp.float32),
        pltpu.VMEM((_C, 16), jnp.float32),
        pltpu.VMEM((_C * 12,), jnp.float32),
        pltpu.SemaphoreType.DMA,
        pltpu.SemaphoreType.DMA,
    ],
)(_sc_body)


def kernel(x, edges, A, b):
  B, N, _ = x.shape
  E = edges.shape[0]
  aw = jnp.concatenate((A, b), axis=3).reshape(N, 12)
  wpad = jnp.zeros((_NPAD, 16), jnp.float32).at[:N, :12].set(aw)
  xpad = jnp.zeros((_NPAD, 3), jnp.float32).at[:N].set(x[0])
  e0 = edges[:, 0].astype(jnp.int32)
  e1 = edges[:, 1].astype(jnp.int32)

  nvf, st = _sc_kernel(wpad, xpad, e0, e1)
  new_vertices = nvf[:N * 3].reshape(B, N, 3)
  stiffness = st.reshape(B, E, 3, 4)
  return (new_vertices, stiffness)


# speedup vs baseline: 5.0569x; 5.0569x over previous
import functools

import jax
import jax.numpy as jnp
from jax import lax
from jax.experimental import pallas as pl
from jax.experimental.pallas import tpu as pltpu
from jax.experimental.pallas import tpu_sc as plsc

_NC = 2
_NS = 16
_NW = _NC * _NS
_L = 16

_N = 50000
_E = 800000
_GN = 1568
_NPAD = _NW * _GN
_EW = _E // _NW
_C = 1000
_NCHUNK = _EW // _C


def _sc_body(w_hbm, x_hbm, e0_hbm, e1_hbm, nv_hbm, st_hbm,
             wv, xv, nvf, idx0, idx1, r0, r1, obf, sem0, sem1):
  wid = lax.axis_index("s") * _NC + lax.axis_index("c")
  lane = lax.iota(jnp.int32, _L)

  nbase = wid * _GN
  pltpu.sync_copy(w_hbm.at[pl.ds(nbase, _GN)], wv)
  pltpu.sync_copy(x_hbm.at[pl.ds(nbase * 3, _GN * 3)], xv)

  def nv_group(g, carry):
    nid = g * _L + lane
    xs = [plsc.load_gather(xv, [nid * 3 + j]) for j in range(3)]
    for i in range(3):
      acc = plsc.load_gather(wv, [nid, jnp.full((_L,), 4 * i + 3, jnp.int32)])
      for j in range(3):
        wij = plsc.load_gather(wv, [nid, jnp.full((_L,), 4 * i + j, jnp.int32)])
        acc = acc + wij * xs[j]
      plsc.store_scatter(nvf, [nid * 3 + i], acc)
    return carry

  lax.fori_loop(0, _GN // _L, nv_group, 0)
  pltpu.sync_copy(nvf, nv_hbm.at[pl.ds(nbase * 3, _GN * 3)])

  ebase = wid * _EW

  def chunk(k, carry):
    cb = ebase + k * _C
    pltpu.sync_copy(e0_hbm.at[pl.ds(cb, _C)], idx0)
    pltpu.sync_copy(e1_hbm.at[pl.ds(cb, _C)], idx1)
    cp0 = pltpu.async_copy(w_hbm.at[idx0], r0, sem0)
    cp1 = pltpu.async_copy(w_hbm.at[idx1], r1, sem1)
    cp0.wait()
    cp1.wait()

    def row(c, rcarry):
      d = r0[c] - r1[c]
      plsc.store_scatter(obf, [c * 12 + lane], d * d, mask=lane < 12)
      return rcarry

    lax.fori_loop(0, _C, row, 0)
    pltpu.sync_copy(obf, st_hbm.at[pl.ds(cb * 12, _C * 12)])
    return carry

  lax.fori_loop(0, _NCHUNK, chunk, 0)


_sc_kernel = functools.partial(
    pl.kernel,
    out_type=(
        jax.ShapeDtypeStruct((_NPAD * 3,), jnp.float32),
        jax.ShapeDtypeStruct((_E * 12,), jnp.float32),
    ),
    mesh=plsc.VectorSubcoreMesh(
        core_axis_name="c", subcore_axis_name="s",
        num_cores=_NC, num_subcores=_NS),
    compiler_params=pltpu.CompilerParams(
        needs_layout_passes=False, use_tc_tiling_on_sc=False),
    scratch_types=[
        pltpu.VMEM((_GN, 16), jnp.float32),
        pltpu.VMEM((_GN * 3,), jnp.float32),
        pltpu.VMEM((_GN * 3,), jnp.float32),
        pltpu.VMEM((_C,), jnp.int32),
        pltpu.VMEM((_C,), jnp.int32),
        pltpu.VMEM((_C, 16), jnp.float32),
        pltpu.VMEM((_C, 16), jnp.float32),
        pltpu.VMEM((_C * 12,), jnp.float32),
        pltpu.SemaphoreType.DMA,
        pltpu.SemaphoreType.DMA,
    ],
)(_sc_body)


def kernel(x, edges, A, b):
  B, N, _ = x.shape
  E = edges.shape[0]
  aw = jnp.concatenate((A, b), axis=3).reshape(N, 12)
  wpad = jnp.zeros((_NPAD, 16), jnp.float32).at[:N, :12].set(aw)
  xpad = jnp.zeros((_NPAD * 3,), jnp.float32).at[:N * 3].set(x[0].reshape(-1))
  e0 = edges[:, 0].astype(jnp.int32)
  e1 = edges[:, 1].astype(jnp.int32)

  nvf, st = _sc_kernel(wpad, xpad, e0, e1)
  new_vertices = nvf[:N * 3].reshape(B, N, 3)
  stiffness = st.reshape(B, E, 3, 4)
  return (new_vertices, stiffness)
